# baseline (device time: 124785 ns/iter reference)
import jax
import jax.numpy as jnp
from jax import lax
from jax.experimental import pallas as pl
from jax.experimental.pallas import tpu as pltpu

N_DEV = 16
M = 2048
N = 1024
CH = M // N_DEV
HALF = N // 2
S = 8
CHS = CH // S
N_HOPS = 2 * (N_DEV - 1)


def kernel(x):
    def body(x_ref, out_ref, comm_f, comm_b, sf_send, sf_recv, sb_send, sb_recv):
        me = lax.axis_index("i")
        left = (me - 1) % N_DEV
        right = (me + 1) % N_DEV

        def fwd_send(h, s, src):
            pltpu.make_async_remote_copy(
                src_ref=src, dst_ref=comm_f.at[h, s],
                send_sem=sf_send.at[h, s], recv_sem=sf_recv.at[h, s],
                device_id=(right,), device_id_type=pl.DeviceIdType.MESH,
            ).start()

        def bwd_send(h, s, src):
            pltpu.make_async_remote_copy(
                src_ref=src, dst_ref=comm_b.at[h, s],
                send_sem=sb_send.at[h, s], recv_sem=sb_recv.at[h, s],
                device_id=(left,), device_id_type=pl.DeviceIdType.MESH,
            ).start()

        def recv_wait(comm, recv_sems, h, s):
            pltpu.make_async_remote_copy(
                src_ref=comm.at[h, s], dst_ref=comm.at[h, s],
                send_sem=sf_send.at[h, s], recv_sem=recv_sems.at[h, s],
                device_id=(right,), device_id_type=pl.DeviceIdType.MESH,
            ).wait_recv()

        barrier_sem = pltpu.get_barrier_semaphore()
        for nbr in (left, right):
            pl.semaphore_signal(
                barrier_sem, inc=1,
                device_id=(nbr,), device_id_type=pl.DeviceIdType.MESH,
            )
        pl.semaphore_wait(barrier_sem, 2)

        out_ref[:, :] = x_ref[0]

        def f_sub(h, s):
            return pl.ds(((me - h) % N_DEV) * CH + s * CHS, CHS)

        def b_sub(h, s):
            return pl.ds(((me + h) % N_DEV) * CH + s * CHS, CHS)

        for s in range(S):
            fwd_send(0, s, out_ref.at[f_sub(0, s), 0:HALF])
            bwd_send(0, s, out_ref.at[b_sub(0, s), HALF:N])
        for h in range(N_DEV - 1):
            for s in range(S):
                recv_wait(comm_f, sf_recv, h, s)
                out_ref[f_sub(h + 1, s), 0:HALF] += comm_f[h, s]
                if h < N_DEV - 2:
                    fwd_send(h + 1, s, out_ref.at[f_sub(h + 1, s), 0:HALF])
                recv_wait(comm_b, sb_recv, h, s)
                out_ref[b_sub(h + 1, s), HALF:N] += comm_b[h, s]
                if h < N_DEV - 2:
                    bwd_send(h + 1, s, out_ref.at[b_sub(h + 1, s), HALF:N])

        AG = N_DEV - 1
        for s in range(S):
            fwd_send(AG, s, out_ref.at[f_sub(AG, s), 0:HALF])
            bwd_send(AG, s, out_ref.at[b_sub(AG, s), HALF:N])
        for h in range(N_DEV - 1):
            for s in range(S):
                recv_wait(comm_f, sf_recv, AG + h, s)
                if h < N_DEV - 2:
                    fwd_send(AG + h + 1, s, comm_f.at[AG + h, s])
                out_ref[f_sub(h, s), 0:HALF] = comm_f[AG + h, s]
                recv_wait(comm_b, sb_recv, AG + h, s)
                if h < N_DEV - 2:
                    bwd_send(AG + h + 1, s, comm_b.at[AG + h, s])
                out_ref[b_sub(h, s), HALF:N] = comm_b[AG + h, s]

        for h in range(N_HOPS):
            for s in range(S):
                for comm, ssem, rsem, dev in (
                    (comm_f, sf_send, sf_recv, right),
                    (comm_b, sb_send, sb_recv, left),
                ):
                    pltpu.make_async_remote_copy(
                        src_ref=comm.at[h, s], dst_ref=comm.at[h, s],
                        send_sem=ssem.at[h, s], recv_sem=rsem.at[h, s],
                        device_id=(dev,), device_id_type=pl.DeviceIdType.MESH,
                    ).wait_send()

    return pl.pallas_call(
        body,
        out_shape=jax.ShapeDtypeStruct((M, N), jnp.float32),
        in_specs=[pl.BlockSpec(memory_space=pltpu.VMEM)],
        out_specs=pl.BlockSpec(memory_space=pltpu.VMEM),
        scratch_shapes=[
            pltpu.VMEM((N_HOPS, S, CHS, HALF), jnp.float32),
            pltpu.VMEM((N_HOPS, S, CHS, HALF), jnp.float32),
            pltpu.SemaphoreType.DMA((N_HOPS, S)),
            pltpu.SemaphoreType.DMA((N_HOPS, S)),
            pltpu.SemaphoreType.DMA((N_HOPS, S)),
            pltpu.SemaphoreType.DMA((N_HOPS, S)),
        ],
        compiler_params=pltpu.CompilerParams(collective_id=0),
    )(x)


# device time: 104177 ns/iter; 1.1978x vs baseline; 1.1978x over previous
import jax
import jax.numpy as jnp
from jax import lax
from jax.experimental import pallas as pl
from jax.experimental.pallas import tpu as pltpu

N_DEV = 16
M = 2048
N = 1024
P_CH = 512
Z_CH = 128
B = 4
BLK = (N // 2) // B
S1 = 4
R1 = P_CH // S1


def kernel(x):
    def body(x_ref, out_ref, c1f, c1b, c2f, c2b, c3f, c3b, c4f, c4b, *sems):
        (s1f_s, s1f_r, s1b_s, s1b_r,
         s2f_s, s2f_r, s2b_s, s2b_r,
         s3f_s, s3f_r, s3b_s, s3b_r,
         s4f_s, s4f_r, s4b_s, s4b_r) = sems

        me = lax.axis_index("i")
        p = me % 4
        z = me // 4
        pf = (p + 1) % 4
        pb = (p - 1) % 4
        p_right = z * 4 + (p + 1) % 4
        p_left = z * 4 + (p - 1) % 4
        z_right = ((z + 1) % 4) * 4 + p
        z_left = ((z - 1) % 4) * 4 + p

        def send(src, dst, ssem, rsem, dev):
            pltpu.make_async_remote_copy(
                src_ref=src, dst_ref=dst, send_sem=ssem, recv_sem=rsem,
                device_id=(dev,), device_id_type=pl.DeviceIdType.MESH,
            ).start()

        def wait_r(buf, ssem, rsem):
            pltpu.make_async_remote_copy(
                src_ref=buf, dst_ref=buf, send_sem=ssem, recv_sem=rsem,
                device_id=(p_right,), device_id_type=pl.DeviceIdType.MESH,
            ).wait_recv()

        def wait_s(buf, ssem, rsem):
            pltpu.make_async_remote_copy(
                src_ref=buf, dst_ref=buf, send_sem=ssem, recv_sem=rsem,
                device_id=(p_right,), device_id_type=pl.DeviceIdType.MESH,
            ).wait_send()

        def fcols(b):
            return pl.ds(b * BLK, BLK)

        def bcols(b):
            return pl.ds(N // 2 + b * BLK, BLK)

        def prow(c, s):
            return pl.ds((c % 4) * P_CH + s * R1, R1)

        def zrow(pc, zz):
            return pl.ds((pc % 4) * P_CH + (zz % 4) * Z_CH, Z_CH)

        def ph1_pro(b):
            for s in range(S1):
                send(x_ref.at[0, prow(p, s), fcols(b)], c1f.at[b, 0, s],
                     s1f_s.at[b, 0, s], s1f_r.at[b, 0, s], p_right)
                send(x_ref.at[0, prow(p, s), bcols(b)], c1b.at[b, 0, s],
                     s1b_s.at[b, 0, s], s1b_r.at[b, 0, s], p_left)

        def ph1_step(b, h):
            for s in range(S1):
                wait_r(c1f.at[b, h, s], s1f_s.at[b, h, s], s1f_r.at[b, h, s])
                out_ref[prow(p - 1 - h, s), fcols(b)] = (
                    x_ref[0, prow(p - 1 - h, s), fcols(b)] + c1f[b, h, s])
                if h < 2:
                    send(out_ref.at[prow(p - 1 - h, s), fcols(b)],
                         c1f.at[b, h + 1, s],
                         s1f_s.at[b, h + 1, s], s1f_r.at[b, h + 1, s], p_right)
                wait_r(c1b.at[b, h, s], s1b_s.at[b, h, s], s1b_r.at[b, h, s])
                out_ref[prow(p + 1 + h, s), bcols(b)] = (
                    x_ref[0, prow(p + 1 + h, s), bcols(b)] + c1b[b, h, s])
                if h < 2:
                    send(out_ref.at[prow(p + 1 + h, s), bcols(b)],
                         c1b.at[b, h + 1, s],
                         s1b_s.at[b, h + 1, s], s1b_r.at[b, h + 1, s], p_left)

        def ph2_pro(b):
            send(out_ref.at[zrow(pf, z), fcols(b)], c2f.at[b, 0],
                 s2f_s.at[b, 0], s2f_r.at[b, 0], z_right)
            send(out_ref.at[zrow(pb, z), bcols(b)], c2b.at[b, 0],
                 s2b_s.at[b, 0], s2b_r.at[b, 0], z_left)

        def ph2_step(b, h):
            wait_r(c2f.at[b, h], s2f_s.at[b, h], s2f_r.at[b, h])
            out_ref[zrow(pf, z - 1 - h), fcols(b)] += c2f[b, h]
            if h < 2:
                send(out_ref.at[zrow(pf, z - 1 - h), fcols(b)], c2f.at[b, h + 1],
                     s2f_s.at[b, h + 1], s2f_r.at[b, h + 1], z_right)
            wait_r(c2b.at[b, h], s2b_s.at[b, h], s2b_r.at[b, h])
            out_ref[zrow(pb, z + 1 + h), bcols(b)] += c2b[b, h]
            if h < 2:
                send(out_ref.at[zrow(pb, z + 1 + h), bcols(b)], c2b.at[b, h + 1],
                     s2b_s.at[b, h + 1], s2b_r.at[b, h + 1], z_left)

        def ph3_pro(b):
            send(out_ref.at[zrow(pf, z + 1), fcols(b)], c3f.at[b, 0],
                 s3f_s.at[b, 0], s3f_r.at[b, 0], z_right)
            send(out_ref.at[zrow(pb, z - 1), bcols(b)], c3b.at[b, 0],
                 s3b_s.at[b, 0], s3b_r.at[b, 0], z_left)

        def ph3_step(b, h):
            wait_r(c3f.at[b, h], s3f_s.at[b, h], s3f_r.at[b, h])
            if h < 2:
                send(c3f.at[b, h], c3f.at[b, h + 1],
                     s3f_s.at[b, h + 1], s3f_r.at[b, h + 1], z_right)
            out_ref[zrow(pf, z - h), fcols(b)] = c3f[b, h]
            wait_r(c3b.at[b, h], s3b_s.at[b, h], s3b_r.at[b, h])
            if h < 2:
                send(c3b.at[b, h], c3b.at[b, h + 1],
                     s3b_s.at[b, h + 1], s3b_r.at[b, h + 1], z_left)
            out_ref[zrow(pb, z + h), bcols(b)] = c3b[b, h]

        def ph4_pro(b):
            for s in range(S1):
                send(out_ref.at[prow(pf, s), fcols(b)], c4f.at[b, 0, s],
                     s4f_s.at[b, 0, s], s4f_r.at[b, 0, s], p_right)
                send(out_ref.at[prow(pb, s), bcols(b)], c4b.at[b, 0, s],
                     s4b_s.at[b, 0, s], s4b_r.at[b, 0, s], p_left)

        def ph4_step(b, h):
            for s in range(S1):
                wait_r(c4f.at[b, h, s], s4f_s.at[b, h, s], s4f_r.at[b, h, s])
                if h < 2:
                    send(c4f.at[b, h, s], c4f.at[b, h + 1, s],
                         s4f_s.at[b, h + 1, s], s4f_r.at[b, h + 1, s], p_right)
                out_ref[prow(p - h, s), fcols(b)] = c4f[b, h, s]
                wait_r(c4b.at[b, h, s], s4b_s.at[b, h, s], s4b_r.at[b, h, s])
                if h < 2:
                    send(c4b.at[b, h, s], c4b.at[b, h + 1, s],
                         s4b_s.at[b, h + 1, s], s4b_r.at[b, h + 1, s], p_left)
                out_ref[prow(p + h, s), bcols(b)] = c4b[b, h, s]

        barrier_sem = pltpu.get_barrier_semaphore()
        for nbr in (p_left, p_right, z_left, z_right):
            pl.semaphore_signal(
                barrier_sem, inc=1,
                device_id=(nbr,), device_id_type=pl.DeviceIdType.MESH,
            )
        pl.semaphore_wait(barrier_sem, 4)

        PROS = (ph1_pro, ph2_pro, ph3_pro, ph4_pro)
        STEPS = (ph1_step, ph2_step, ph3_step, ph4_step)
        for t in range(B + 3):
            active = [(t - b, b) for b in range(B) if 0 <= t - b <= 3]
            for k, b in active:
                PROS[k](b)
            for h in range(3):
                for k, b in active:
                    STEPS[k](b, h)

        for b in range(B):
            for h in range(3):
                for s in range(S1):
                    wait_s(c1f.at[b, h, s], s1f_s.at[b, h, s], s1f_r.at[b, h, s])
                    wait_s(c1b.at[b, h, s], s1b_s.at[b, h, s], s1b_r.at[b, h, s])
                    wait_s(c4f.at[b, h, s], s4f_s.at[b, h, s], s4f_r.at[b, h, s])
                    wait_s(c4b.at[b, h, s], s4b_s.at[b, h, s], s4b_r.at[b, h, s])
                wait_s(c2f.at[b, h], s2f_s.at[b, h], s2f_r.at[b, h])
                wait_s(c2b.at[b, h], s2b_s.at[b, h], s2b_r.at[b, h])
                wait_s(c3f.at[b, h], s3f_s.at[b, h], s3f_r.at[b, h])
                wait_s(c3b.at[b, h], s3b_s.at[b, h], s3b_r.at[b, h])

    plane_buf = pltpu.VMEM((B, 3, S1, R1, BLK), jnp.float32)
    z_buf = pltpu.VMEM((B, 3, Z_CH, BLK), jnp.float32)
    plane_sem = pltpu.SemaphoreType.DMA((B, 3, S1))
    z_sem = pltpu.SemaphoreType.DMA((B, 3))
    return pl.pallas_call(
        body,
        out_shape=jax.ShapeDtypeStruct((M, N), jnp.float32),
        in_specs=[pl.BlockSpec(memory_space=pltpu.VMEM)],
        out_specs=pl.BlockSpec(memory_space=pltpu.VMEM),
        scratch_shapes=[
            plane_buf, plane_buf, z_buf, z_buf, z_buf, z_buf,
            plane_buf, plane_buf,
            plane_sem, plane_sem, plane_sem, plane_sem,
            z_sem, z_sem, z_sem, z_sem,
            z_sem, z_sem, z_sem, z_sem,
            plane_sem, plane_sem, plane_sem, plane_sem,
        ],
        compiler_params=pltpu.CompilerParams(collective_id=0),
    )(x)
